# baseline (speedup 1.0000x reference)
import functools

import jax
import jax.numpy as jnp
from jax import lax
from jax.experimental import pallas as pl
from jax.experimental.pallas import tpu as pltpu
from jax.experimental.pallas import tpu_sc as plsc

LEVELS = 256
TIMESTAMPS = 4096
DIM = 10000
N = 4096

NC = 2
NS = 16
L = 16
NW = NC * NS
N_SC = 2048
N_TC = N - N_SC
SPW = N_SC // NW
CHUNKS = DIM // L

SUB = 80
LANE = 125

W2 = 400
NACT2 = DIM // W2

_MESH = plsc.VectorSubcoreMesh(
    core_axis_name="c", subcore_axis_name="s", num_cores=NC, num_subcores=NS
)


def _worker_id():
    return lax.axis_index("s") * NC + lax.axis_index("c")


@functools.partial(
    pl.kernel,
    out_type=jax.ShapeDtypeStruct((NW, DIM), jnp.float32),
    mesh=_MESH,
    compiler_params=pltpu.CompilerParams(use_tc_tiling_on_sc=False),
    scratch_types=[
        pltpu.VMEM((SPW, 1), jnp.int32),
        pltpu.VMEM((SPW, 1), jnp.int32),
        pltpu.VMEM((SPW, 1), jnp.int32),
        pltpu.VMEM((SPW, 1), jnp.int32),
        pltpu.VMEM((1, DIM), jnp.float32),
        pltpu.VMEM((1, DIM), jnp.float32),
        pltpu.VMEM((1, DIM), jnp.float32),
        pltpu.VMEM((1, DIM), jnp.float32),
        pltpu.VMEM((1, DIM), jnp.float32),
        pltpu.VMEM((1, DIM), jnp.float32),
        pltpu.VMEM((1, DIM), jnp.float32),
        pltpu.VMEM((1, DIM), jnp.float32),
        pltpu.VMEM((DIM,), jnp.float32),
        pltpu.SemaphoreType.DMA,
        pltpu.SemaphoreType.DMA,
    ],
)
def _phase1(ti, xi, yi, zi, Wt, Wx, Wy, Wz, part,
            ti_v, xi_v, yi_v, zi_v,
            wt_a, wx_a, wy_a, wz_a, wt_b, wx_b, wy_b, wz_b,
            acc, sem_a, sem_b):
    wid = _worker_id()
    base = wid * SPW
    pltpu.sync_copy(ti.at[pl.ds(base, SPW)], ti_v)
    pltpu.sync_copy(xi.at[pl.ds(base, SPW)], xi_v)
    pltpu.sync_copy(yi.at[pl.ds(base, SPW)], yi_v)
    pltpu.sync_copy(zi.at[pl.ds(base, SPW)], zi_v)

    @plsc.parallel_loop(0, CHUNKS, unroll=25)
    def _zero(i):
        acc[pl.ds(i * L, L)] = jnp.zeros((L,), jnp.float32)

    set_a = (wt_a, wx_a, wy_a, wz_a)
    set_b = (wt_b, wx_b, wy_b, wz_b)

    def fire(s, bufs, sem):
        pltpu.async_copy(Wt.at[ti_v.at[s]], bufs[0], sem)
        pltpu.async_copy(Wx.at[xi_v.at[s]], bufs[1], sem)
        pltpu.async_copy(Wy.at[yi_v.at[s]], bufs[2], sem)
        pltpu.async_copy(Wz.at[zi_v.at[s]], bufs[3], sem)

    def drain(bufs, sem):
        dummy = Wt.at[pl.ds(0, 1)]
        pltpu.make_async_copy(dummy, bufs[0], sem).wait()
        pltpu.make_async_copy(dummy, bufs[1], sem).wait()
        pltpu.make_async_copy(dummy, bufs[2], sem).wait()
        pltpu.make_async_copy(dummy, bufs[3], sem).wait()

    def accumulate(bufs):
        @plsc.parallel_loop(0, CHUNKS, unroll=25)
        def _chunk(i):
            sl = pl.ds(i * L, L)
            p = bufs[0][0, sl] * bufs[1][0, sl]
            p = p * bufs[2][0, sl]
            p = p * bufs[3][0, sl]
            plsc.addupdate(acc.at[sl], p)

    fire(0, set_a, sem_a)

    def pair_body(p, carry):
        s = 2 * p
        fire(s + 1, set_b, sem_b)
        drain(set_a, sem_a)
        accumulate(set_a)

        @pl.when(p < SPW // 2 - 1)
        def _():
            fire(s + 2, set_a, sem_a)

        drain(set_b, sem_b)
        accumulate(set_b)
        return carry

    lax.fori_loop(0, SPW // 2, pair_body, 0)
    pltpu.sync_copy(acc, part.at[wid])


def _tc_body(ti, xi, yi, zi, wt, wx, wy, wz, o_ref):
    i = pl.program_id(0)

    @pl.when(i == 0)
    def _():
        o_ref[...] = jnp.zeros_like(o_ref)

    o_ref[...] += wt[...] * wx[...] * wy[...] * wz[...]


_tc_partial = pl.pallas_call(
    _tc_body,
    grid_spec=pltpu.PrefetchScalarGridSpec(
        num_scalar_prefetch=4,
        grid=(N_TC,),
        in_specs=[
            pl.BlockSpec((1, SUB, LANE), lambda i, ti, xi, yi, zi: (ti[i], 0, 0)),
            pl.BlockSpec((1, SUB, LANE), lambda i, ti, xi, yi, zi: (xi[i], 0, 0)),
            pl.BlockSpec((1, SUB, LANE), lambda i, ti, xi, yi, zi: (yi[i], 0, 0)),
            pl.BlockSpec((1, SUB, LANE), lambda i, ti, xi, yi, zi: (zi[i], 0, 0)),
        ],
        out_specs=pl.BlockSpec((1, SUB, LANE), lambda i, *_: (0, 0, 0)),
    ),
    out_shape=jax.ShapeDtypeStruct((1, SUB, LANE), jnp.float32),
)


@functools.partial(
    pl.kernel,
    out_type=jax.ShapeDtypeStruct((DIM,), jnp.float32),
    mesh=_MESH,
    compiler_params=pltpu.CompilerParams(use_tc_tiling_on_sc=False),
    scratch_types=[
        pltpu.VMEM((NW, W2), jnp.float32),
        pltpu.VMEM((W2,), jnp.float32),
        pltpu.VMEM((W2,), jnp.float32),
    ],
)
def _phase2(part, tcp, out, buf, tbuf, outb):
    wid = _worker_id()

    @pl.when(wid < NACT2)
    def _():
        base = wid * W2
        pltpu.sync_copy(part.at[:, pl.ds(base, W2)], buf)
        pltpu.sync_copy(tcp.at[pl.ds(base, W2)], tbuf)

        def body(i, carry):
            sl = pl.ds(i * L, L)
            a = tbuf[sl]
            for k in range(NW):
                a = a + buf[k, sl]
            e = jnp.exp(a * 2.0)
            outb[sl] = 1.0 - 2.0 / (e + 1.0)
            return carry

        lax.fori_loop(0, W2 // L, body, 0)
        pltpu.sync_copy(outb, out.at[pl.ds(base, W2)])


def _level_idx(value, low, high, n):
    idx = jnp.round((value - low) / (high - low) * (n - 1)).astype(jnp.int32)
    return jnp.clip(idx, 0, n - 1)


def kernel(input, Wt, Wx, Wy, Wz):
    t = input[:, 0] - input[0, 0]
    xi = _level_idx(input[:, 1], 0.0, 1.0, LEVELS)
    yi = _level_idx(input[:, 2], 0.0, 1.0, LEVELS)
    zi = _level_idx(input[:, 3], 0.0, 1.0, LEVELS)
    ti = _level_idx(t, 0.0, float(TIMESTAMPS), TIMESTAMPS)

    part = _phase1(
        ti[:N_SC].reshape(-1, 1), xi[:N_SC].reshape(-1, 1),
        yi[:N_SC].reshape(-1, 1), zi[:N_SC].reshape(-1, 1),
        Wt, Wx, Wy, Wz,
    )
    tcp = _tc_partial(
        ti[N_SC:], xi[N_SC:], yi[N_SC:], zi[N_SC:],
        Wt.reshape(TIMESTAMPS, SUB, LANE), Wx.reshape(LEVELS, SUB, LANE),
        Wy.reshape(LEVELS, SUB, LANE), Wz.reshape(LEVELS, SUB, LANE),
    ).reshape(DIM)
    return _phase2(part, tcp)


# speedup vs baseline: 1.0072x; 1.0072x over previous
import functools

import jax
import jax.numpy as jnp
from jax import lax
from jax.experimental import pallas as pl
from jax.experimental.pallas import tpu as pltpu
from jax.experimental.pallas import tpu_sc as plsc

LEVELS = 256
TIMESTAMPS = 4096
DIM = 10000
N = 4096

NC = 2
NS = 16
L = 16
NW = NC * NS
N_SC = 2048
N_TC = N - N_SC
SPW = N_SC // NW
CHUNKS = DIM // L

SUB = 80
LANE = 125

W2 = 400
NACT2 = DIM // W2

_MESH = plsc.VectorSubcoreMesh(
    core_axis_name="c", subcore_axis_name="s", num_cores=NC, num_subcores=NS
)


def _worker_id():
    return lax.axis_index("s") * NC + lax.axis_index("c")


@functools.partial(
    pl.kernel,
    out_type=jax.ShapeDtypeStruct((NW, DIM), jnp.float32),
    mesh=_MESH,
    compiler_params=pltpu.CompilerParams(use_tc_tiling_on_sc=False),
    scratch_types=[
        pltpu.VMEM((SPW, 1), jnp.int32),
        pltpu.VMEM((SPW, 1), jnp.int32),
        pltpu.VMEM((SPW, 1), jnp.int32),
        pltpu.VMEM((SPW, 1), jnp.int32),
        pltpu.VMEM((1, DIM), jnp.float32),
        pltpu.VMEM((1, DIM), jnp.float32),
        pltpu.VMEM((1, DIM), jnp.float32),
        pltpu.VMEM((1, DIM), jnp.float32),
        pltpu.VMEM((1, DIM), jnp.float32),
        pltpu.VMEM((1, DIM), jnp.float32),
        pltpu.VMEM((1, DIM), jnp.float32),
        pltpu.VMEM((1, DIM), jnp.float32),
        pltpu.VMEM((DIM,), jnp.float32),
        pltpu.SemaphoreType.DMA,
        pltpu.SemaphoreType.DMA,
    ],
)
def _phase1(ti, xi, yi, zi, Wt, Wx, Wy, Wz, part,
            ti_v, xi_v, yi_v, zi_v,
            wt_a, wx_a, wy_a, wz_a, wt_b, wx_b, wy_b, wz_b,
            acc, sem_a, sem_b):
    wid = _worker_id()
    base = wid * SPW
    pltpu.sync_copy(ti.at[pl.ds(base, SPW)], ti_v)
    pltpu.sync_copy(xi.at[pl.ds(base, SPW)], xi_v)
    pltpu.sync_copy(yi.at[pl.ds(base, SPW)], yi_v)
    pltpu.sync_copy(zi.at[pl.ds(base, SPW)], zi_v)

    @plsc.parallel_loop(0, CHUNKS, unroll=25)
    def _zero(i):
        acc[pl.ds(i * L, L)] = jnp.zeros((L,), jnp.float32)

    set_a = (wt_a, wx_a, wy_a, wz_a)
    set_b = (wt_b, wx_b, wy_b, wz_b)

    def fire(s, bufs, sem):
        pltpu.async_copy(Wt.at[ti_v.at[s]], bufs[0], sem)
        pltpu.async_copy(Wx.at[xi_v.at[s]], bufs[1], sem)
        pltpu.async_copy(Wy.at[yi_v.at[s]], bufs[2], sem)
        pltpu.async_copy(Wz.at[zi_v.at[s]], bufs[3], sem)

    def drain(bufs, sem):
        dummy = Wt.at[pl.ds(0, 1)]
        pltpu.make_async_copy(dummy, bufs[0], sem).wait()
        pltpu.make_async_copy(dummy, bufs[1], sem).wait()
        pltpu.make_async_copy(dummy, bufs[2], sem).wait()
        pltpu.make_async_copy(dummy, bufs[3], sem).wait()

    def accumulate(bufs):
        @plsc.parallel_loop(0, CHUNKS, unroll=25)
        def _chunk(i):
            sl = pl.ds(i * L, L)
            p = bufs[0][0, sl] * bufs[1][0, sl]
            p = p * bufs[2][0, sl]
            p = p * bufs[3][0, sl]
            plsc.addupdate(acc.at[sl], p)

    fire(0, set_a, sem_a)

    def pair_body(p, carry):
        s = 2 * p
        fire(s + 1, set_b, sem_b)
        drain(set_a, sem_a)
        accumulate(set_a)

        @pl.when(p < SPW // 2 - 1)
        def _():
            fire(s + 2, set_a, sem_a)

        drain(set_b, sem_b)
        accumulate(set_b)
        return carry

    lax.fori_loop(0, SPW // 2, pair_body, 0)
    pltpu.sync_copy(acc, part.at[wid])


def _tc_body(ti, xi, yi, zi, wt, wx, wy, wz, o_ref, buf, sems):
    tabs = (wt, wx, wy, wz)
    idxs = (ti, xi, yi, zi)

    def fire(s, slot):
        for k in range(4):
            pltpu.make_async_copy(
                tabs[k].at[idxs[k][s]], buf.at[slot, k], sems.at[slot, k]
            ).start()

    def drain(slot):
        for k in range(4):
            pltpu.make_async_copy(
                tabs[k].at[0], buf.at[slot, k], sems.at[slot, k]
            ).wait()

    def accumulate(slot):
        p = buf[slot, 0] * buf[slot, 1]
        p = p * (buf[slot, 2] * buf[slot, 3])
        o_ref[...] += p

    o_ref[...] = jnp.zeros_like(o_ref)
    fire(0, 0)

    def pair_body(p, carry):
        s = 2 * p
        fire(s + 1, 1)
        drain(0)
        accumulate(0)

        @pl.when(p < N_TC // 2 - 1)
        def _():
            fire(s + 2, 0)

        drain(1)
        accumulate(1)
        return carry

    lax.fori_loop(0, N_TC // 2, pair_body, 0)


_tc_partial = pl.pallas_call(
    _tc_body,
    grid_spec=pltpu.PrefetchScalarGridSpec(
        num_scalar_prefetch=4,
        grid=(1,),
        in_specs=[
            pl.BlockSpec(memory_space=pl.ANY),
            pl.BlockSpec(memory_space=pl.ANY),
            pl.BlockSpec(memory_space=pl.ANY),
            pl.BlockSpec(memory_space=pl.ANY),
        ],
        out_specs=pl.BlockSpec((SUB, LANE), lambda i, *_: (0, 0)),
        scratch_shapes=[
            pltpu.VMEM((2, 4, SUB, LANE), jnp.float32),
            pltpu.SemaphoreType.DMA((2, 4)),
        ],
    ),
    out_shape=jax.ShapeDtypeStruct((SUB, LANE), jnp.float32),
)


@functools.partial(
    pl.kernel,
    out_type=jax.ShapeDtypeStruct((DIM,), jnp.float32),
    mesh=_MESH,
    compiler_params=pltpu.CompilerParams(use_tc_tiling_on_sc=False),
    scratch_types=[
        pltpu.VMEM((NW, W2), jnp.float32),
        pltpu.VMEM((W2,), jnp.float32),
        pltpu.VMEM((W2,), jnp.float32),
    ],
)
def _phase2(part, tcp, out, buf, tbuf, outb):
    wid = _worker_id()

    @pl.when(wid < NACT2)
    def _():
        base = wid * W2
        pltpu.sync_copy(part.at[:, pl.ds(base, W2)], buf)
        pltpu.sync_copy(tcp.at[pl.ds(base, W2)], tbuf)

        def body(i, carry):
            sl = pl.ds(i * L, L)
            a = tbuf[sl]
            for k in range(NW):
                a = a + buf[k, sl]
            e = jnp.exp(a * 2.0)
            outb[sl] = 1.0 - 2.0 / (e + 1.0)
            return carry

        lax.fori_loop(0, W2 // L, body, 0)
        pltpu.sync_copy(outb, out.at[pl.ds(base, W2)])


def _level_idx(value, low, high, n):
    idx = jnp.round((value - low) / (high - low) * (n - 1)).astype(jnp.int32)
    return jnp.clip(idx, 0, n - 1)


def kernel(input, Wt, Wx, Wy, Wz):
    t = input[:, 0] - input[0, 0]
    xi = _level_idx(input[:, 1], 0.0, 1.0, LEVELS)
    yi = _level_idx(input[:, 2], 0.0, 1.0, LEVELS)
    zi = _level_idx(input[:, 3], 0.0, 1.0, LEVELS)
    ti = _level_idx(t, 0.0, float(TIMESTAMPS), TIMESTAMPS)

    part = _phase1(
        ti[:N_SC].reshape(-1, 1), xi[:N_SC].reshape(-1, 1),
        yi[:N_SC].reshape(-1, 1), zi[:N_SC].reshape(-1, 1),
        Wt, Wx, Wy, Wz,
    )
    tcp = _tc_partial(
        ti[N_SC:], xi[N_SC:], yi[N_SC:], zi[N_SC:],
        Wt.reshape(TIMESTAMPS, SUB, LANE), Wx.reshape(LEVELS, SUB, LANE),
        Wy.reshape(LEVELS, SUB, LANE), Wz.reshape(LEVELS, SUB, LANE),
    ).reshape(DIM)
    return _phase2(part, tcp)


# speedup vs baseline: 2.6738x; 2.6546x over previous
import functools

import jax
import jax.numpy as jnp
from jax import lax
from jax.experimental import pallas as pl
from jax.experimental.pallas import tpu as pltpu
from jax.experimental.pallas import tpu_sc as plsc

LEVELS = 256
TIMESTAMPS = 4096
DIM = 10000
N = 4096

NC = 2
NS = 16
L = 16
NW = NC * NS
N_SC = 2048
N_TC = N - N_SC
SPW = N_SC // NW
CHUNKS = DIM // L

B_TC = 8
G_TC = N_TC // B_TC

W2 = 400
NACT2 = DIM // W2

_MESH = plsc.VectorSubcoreMesh(
    core_axis_name="c", subcore_axis_name="s", num_cores=NC, num_subcores=NS
)


def _worker_id():
    return lax.axis_index("s") * NC + lax.axis_index("c")


@functools.partial(
    pl.kernel,
    out_type=jax.ShapeDtypeStruct((NW, DIM), jnp.float32),
    mesh=_MESH,
    compiler_params=pltpu.CompilerParams(use_tc_tiling_on_sc=False),
    scratch_types=[
        pltpu.VMEM((SPW, 1), jnp.int32),
        pltpu.VMEM((SPW, 1), jnp.int32),
        pltpu.VMEM((SPW, 1), jnp.int32),
        pltpu.VMEM((SPW, 1), jnp.int32),
        pltpu.VMEM((1, DIM), jnp.float32),
        pltpu.VMEM((1, DIM), jnp.float32),
        pltpu.VMEM((1, DIM), jnp.float32),
        pltpu.VMEM((1, DIM), jnp.float32),
        pltpu.VMEM((1, DIM), jnp.float32),
        pltpu.VMEM((1, DIM), jnp.float32),
        pltpu.VMEM((1, DIM), jnp.float32),
        pltpu.VMEM((1, DIM), jnp.float32),
        pltpu.VMEM((DIM,), jnp.float32),
        pltpu.SemaphoreType.DMA,
        pltpu.SemaphoreType.DMA,
    ],
)
def _phase1(ti, xi, yi, zi, Wt, Wx, Wy, Wz, part,
            ti_v, xi_v, yi_v, zi_v,
            wt_a, wx_a, wy_a, wz_a, wt_b, wx_b, wy_b, wz_b,
            acc, sem_a, sem_b):
    wid = _worker_id()
    base = wid * SPW
    pltpu.sync_copy(ti.at[pl.ds(base, SPW)], ti_v)
    pltpu.sync_copy(xi.at[pl.ds(base, SPW)], xi_v)
    pltpu.sync_copy(yi.at[pl.ds(base, SPW)], yi_v)
    pltpu.sync_copy(zi.at[pl.ds(base, SPW)], zi_v)

    @plsc.parallel_loop(0, CHUNKS, unroll=25)
    def _zero(i):
        acc[pl.ds(i * L, L)] = jnp.zeros((L,), jnp.float32)

    set_a = (wt_a, wx_a, wy_a, wz_a)
    set_b = (wt_b, wx_b, wy_b, wz_b)

    def fire(s, bufs, sem):
        pltpu.async_copy(Wt.at[ti_v.at[s]], bufs[0], sem)
        pltpu.async_copy(Wx.at[xi_v.at[s]], bufs[1], sem)
        pltpu.async_copy(Wy.at[yi_v.at[s]], bufs[2], sem)
        pltpu.async_copy(Wz.at[zi_v.at[s]], bufs[3], sem)

    def drain(bufs, sem):
        dummy = Wt.at[pl.ds(0, 1)]
        pltpu.make_async_copy(dummy, bufs[0], sem).wait()
        pltpu.make_async_copy(dummy, bufs[1], sem).wait()
        pltpu.make_async_copy(dummy, bufs[2], sem).wait()
        pltpu.make_async_copy(dummy, bufs[3], sem).wait()

    def accumulate(bufs):
        @plsc.parallel_loop(0, CHUNKS, unroll=25)
        def _chunk(i):
            sl = pl.ds(i * L, L)
            p = bufs[0][0, sl] * bufs[1][0, sl]
            p = p * bufs[2][0, sl]
            p = p * bufs[3][0, sl]
            plsc.addupdate(acc.at[sl], p)

    fire(0, set_a, sem_a)

    def pair_body(p, carry):
        s = 2 * p
        fire(s + 1, set_b, sem_b)
        drain(set_a, sem_a)
        accumulate(set_a)

        @pl.when(p < SPW // 2 - 1)
        def _():
            fire(s + 2, set_a, sem_a)

        drain(set_b, sem_b)
        accumulate(set_b)
        return carry

    lax.fori_loop(0, SPW // 2, pair_body, 0)
    pltpu.sync_copy(acc, part.at[wid])


def _tc_body(ti, xi, yi, zi, wt, wx, wy, wz, o_ref, buf, sems):
    tabs = (wt, wx, wy, wz)
    idxs = (ti, xi, yi, zi)

    def fire(g, slot):
        for k in range(4):
            for j in range(B_TC):
                pltpu.make_async_copy(
                    tabs[k].at[idxs[k][g * B_TC + j]],
                    buf.at[slot, k, j],
                    sems.at[slot, k],
                ).start()

    def drain(slot):
        for k in range(4):
            for j in range(B_TC):
                pltpu.make_async_copy(
                    tabs[k].at[0], buf.at[slot, k, j], sems.at[slot, k]
                ).wait()

    def accumulate(slot):
        p = buf[slot, 0] * buf[slot, 1]
        p = p * (buf[slot, 2] * buf[slot, 3])
        o_ref[...] += p

    o_ref[...] = jnp.zeros_like(o_ref)
    fire(0, 0)

    def pair_body(p, carry):
        g = 2 * p
        fire(g + 1, 1)
        drain(0)
        accumulate(0)

        @pl.when(p < G_TC // 2 - 1)
        def _():
            fire(g + 2, 0)

        drain(1)
        accumulate(1)
        return carry

    lax.fori_loop(0, G_TC // 2, pair_body, 0)


_tc_partial = pl.pallas_call(
    _tc_body,
    grid_spec=pltpu.PrefetchScalarGridSpec(
        num_scalar_prefetch=4,
        grid=(1,),
        in_specs=[
            pl.BlockSpec(memory_space=pl.ANY),
            pl.BlockSpec(memory_space=pl.ANY),
            pl.BlockSpec(memory_space=pl.ANY),
            pl.BlockSpec(memory_space=pl.ANY),
        ],
        out_specs=pl.BlockSpec((B_TC, DIM), lambda i, *_: (0, 0)),
        scratch_shapes=[
            pltpu.VMEM((2, 4, B_TC, DIM), jnp.float32),
            pltpu.SemaphoreType.DMA((2, 4)),
        ],
    ),
    out_shape=jax.ShapeDtypeStruct((B_TC, DIM), jnp.float32),
)


@functools.partial(
    pl.kernel,
    out_type=jax.ShapeDtypeStruct((DIM,), jnp.float32),
    mesh=_MESH,
    compiler_params=pltpu.CompilerParams(use_tc_tiling_on_sc=False),
    scratch_types=[
        pltpu.VMEM((NW, W2), jnp.float32),
        pltpu.VMEM((B_TC, W2), jnp.float32),
        pltpu.VMEM((W2,), jnp.float32),
    ],
)
def _phase2(part, tcp, out, buf, tbuf, outb):
    wid = _worker_id()

    @pl.when(wid < NACT2)
    def _():
        base = wid * W2
        pltpu.sync_copy(part.at[:, pl.ds(base, W2)], buf)
        pltpu.sync_copy(tcp.at[:, pl.ds(base, W2)], tbuf)

        def body(i, carry):
            sl = pl.ds(i * L, L)
            a = tbuf[0, sl]
            for k in range(1, B_TC):
                a = a + tbuf[k, sl]
            for k in range(NW):
                a = a + buf[k, sl]
            e = jnp.exp(a * 2.0)
            outb[sl] = 1.0 - 2.0 / (e + 1.0)
            return carry

        lax.fori_loop(0, W2 // L, body, 0)
        pltpu.sync_copy(outb, out.at[pl.ds(base, W2)])


def _level_idx(value, low, high, n):
    idx = jnp.round((value - low) / (high - low) * (n - 1)).astype(jnp.int32)
    return jnp.clip(idx, 0, n - 1)


def kernel(input, Wt, Wx, Wy, Wz):
    t = input[:, 0] - input[0, 0]
    xi = _level_idx(input[:, 1], 0.0, 1.0, LEVELS)
    yi = _level_idx(input[:, 2], 0.0, 1.0, LEVELS)
    zi = _level_idx(input[:, 3], 0.0, 1.0, LEVELS)
    ti = _level_idx(t, 0.0, float(TIMESTAMPS), TIMESTAMPS)

    part = _phase1(
        ti[:N_SC].reshape(-1, 1), xi[:N_SC].reshape(-1, 1),
        yi[:N_SC].reshape(-1, 1), zi[:N_SC].reshape(-1, 1),
        Wt, Wx, Wy, Wz,
    )
    tcp = _tc_partial(
        ti[N_SC:], xi[N_SC:], yi[N_SC:], zi[N_SC:], Wt, Wx, Wy, Wz
    )
    return _phase2(part, tcp)


# speedup vs baseline: 2.6753x; 1.0006x over previous
import functools

import jax
import jax.numpy as jnp
from jax import lax
from jax.experimental import pallas as pl
from jax.experimental.pallas import tpu as pltpu
from jax.experimental.pallas import tpu_sc as plsc

LEVELS = 256
TIMESTAMPS = 4096
DIM = 10000
N = 4096

NC = 2
NS = 16
L = 16
NW = NC * NS
N_SC = 2048
N_TC = N - N_SC
SPW = N_SC // NW
CHUNKS = DIM // L

B_TC = 8
G_TC = N_TC // B_TC

W2 = 400
NACT2 = DIM // W2

_MESH = plsc.VectorSubcoreMesh(
    core_axis_name="c", subcore_axis_name="s", num_cores=NC, num_subcores=NS
)


def _worker_id():
    return lax.axis_index("s") * NC + lax.axis_index("c")


@functools.partial(
    pl.kernel,
    out_type=jax.ShapeDtypeStruct((NW, DIM), jnp.float32),
    mesh=_MESH,
    compiler_params=pltpu.CompilerParams(use_tc_tiling_on_sc=False),
    scratch_types=[
        pltpu.VMEM((SPW, 1), jnp.int32),
        pltpu.VMEM((SPW, 1), jnp.int32),
        pltpu.VMEM((SPW, 1), jnp.int32),
        pltpu.VMEM((SPW, 1), jnp.int32),
        pltpu.VMEM((1, DIM), jnp.float32),
        pltpu.VMEM((1, DIM), jnp.float32),
        pltpu.VMEM((1, DIM), jnp.float32),
        pltpu.VMEM((1, DIM), jnp.float32),
        pltpu.VMEM((1, DIM), jnp.float32),
        pltpu.VMEM((1, DIM), jnp.float32),
        pltpu.VMEM((1, DIM), jnp.float32),
        pltpu.VMEM((1, DIM), jnp.float32),
        pltpu.VMEM((DIM,), jnp.float32),
        pltpu.SemaphoreType.DMA,
        pltpu.SemaphoreType.DMA,
    ],
)
def _phase1(ti, xi, yi, zi, Wt, Wx, Wy, Wz, part,
            ti_v, xi_v, yi_v, zi_v,
            wt_a, wx_a, wy_a, wz_a, wt_b, wx_b, wy_b, wz_b,
            acc, sem_a, sem_b):
    wid = _worker_id()
    base = wid * SPW
    pltpu.sync_copy(ti.at[pl.ds(base, SPW)], ti_v)
    pltpu.sync_copy(xi.at[pl.ds(base, SPW)], xi_v)
    pltpu.sync_copy(yi.at[pl.ds(base, SPW)], yi_v)
    pltpu.sync_copy(zi.at[pl.ds(base, SPW)], zi_v)

    @plsc.parallel_loop(0, CHUNKS, unroll=25)
    def _zero(i):
        acc[pl.ds(i * L, L)] = jnp.zeros((L,), jnp.float32)

    set_a = (wt_a, wx_a, wy_a, wz_a)
    set_b = (wt_b, wx_b, wy_b, wz_b)

    def fire(s, bufs, sem):
        pltpu.async_copy(Wt.at[ti_v.at[s]], bufs[0], sem)
        pltpu.async_copy(Wx.at[xi_v.at[s]], bufs[1], sem)
        pltpu.async_copy(Wy.at[yi_v.at[s]], bufs[2], sem)
        pltpu.async_copy(Wz.at[zi_v.at[s]], bufs[3], sem)

    def drain(bufs, sem):
        dummy = Wt.at[pl.ds(0, 1)]
        pltpu.make_async_copy(dummy, bufs[0], sem).wait()
        pltpu.make_async_copy(dummy, bufs[1], sem).wait()
        pltpu.make_async_copy(dummy, bufs[2], sem).wait()
        pltpu.make_async_copy(dummy, bufs[3], sem).wait()

    def accumulate(bufs):
        @plsc.parallel_loop(0, CHUNKS, unroll=25)
        def _chunk(i):
            sl = pl.ds(i * L, L)
            p = bufs[0][0, sl] * bufs[1][0, sl]
            p = p * bufs[2][0, sl]
            p = p * bufs[3][0, sl]
            plsc.addupdate(acc.at[sl], p)

    fire(0, set_a, sem_a)

    def pair_body(p, carry):
        s = 2 * p
        fire(s + 1, set_b, sem_b)
        drain(set_a, sem_a)
        accumulate(set_a)

        @pl.when(p < SPW // 2 - 1)
        def _():
            fire(s + 2, set_a, sem_a)

        drain(set_b, sem_b)
        accumulate(set_b)
        return carry

    lax.fori_loop(0, SPW // 2, pair_body, 0)
    pltpu.sync_copy(acc, part.at[wid])


def _tc_body(ti, xi, yi, zi, wt, wx, wy, wz, o_ref, buf, sems):
    tabs = (wt, wx, wy, wz)
    idxs = (ti, xi, yi, zi)

    def fire(g, slot):
        for k in range(4):
            for j in range(B_TC):
                pltpu.make_async_copy(
                    tabs[k].at[idxs[k][g * B_TC + j]],
                    buf.at[slot, k, j],
                    sems.at[slot, k],
                ).start()

    def drain(slot):
        for k in range(4):
            for j in range(B_TC):
                pltpu.make_async_copy(
                    tabs[k].at[0], buf.at[slot, k, j], sems.at[slot, k]
                ).wait()

    def accumulate(slot):
        p = buf[slot, 0] * buf[slot, 1]
        p = p * (buf[slot, 2] * buf[slot, 3])
        o_ref[...] += p

    o_ref[...] = jnp.zeros_like(o_ref)
    fire(0, 0)

    def pair_body(p, carry):
        g = 2 * p
        fire(g + 1, 1)
        drain(0)
        accumulate(0)

        @pl.when(p < G_TC // 2 - 1)
        def _():
            fire(g + 2, 0)

        drain(1)
        accumulate(1)
        return carry

    lax.fori_loop(0, G_TC // 2, pair_body, 0)


_tc_partial = pl.pallas_call(
    _tc_body,
    grid_spec=pltpu.PrefetchScalarGridSpec(
        num_scalar_prefetch=4,
        grid=(1,),
        in_specs=[
            pl.BlockSpec(memory_space=pl.ANY),
            pl.BlockSpec(memory_space=pl.ANY),
            pl.BlockSpec(memory_space=pl.ANY),
            pl.BlockSpec(memory_space=pl.ANY),
        ],
        out_specs=pl.BlockSpec((B_TC, DIM), lambda i, *_: (0, 0)),
        scratch_shapes=[
            pltpu.VMEM((2, 4, B_TC, DIM), jnp.float32),
            pltpu.SemaphoreType.DMA((2, 4)),
        ],
    ),
    out_shape=jax.ShapeDtypeStruct((B_TC, DIM), jnp.float32),
)


@functools.partial(
    pl.kernel,
    out_type=jax.ShapeDtypeStruct((DIM,), jnp.float32),
    mesh=_MESH,
    compiler_params=pltpu.CompilerParams(use_tc_tiling_on_sc=False),
    scratch_types=[
        pltpu.VMEM((NW, W2), jnp.float32),
        pltpu.VMEM((B_TC, W2), jnp.float32),
        pltpu.VMEM((W2,), jnp.float32),
    ],
)
def _phase2(part, tcp, out, buf, tbuf, outb):
    wid = _worker_id()

    @pl.when(wid < NACT2)
    def _():
        base = wid * W2
        pltpu.sync_copy(part.at[:, pl.ds(base, W2)], buf)
        pltpu.sync_copy(tcp.at[:, pl.ds(base, W2)], tbuf)

        def body(i, carry):
            sl = pl.ds(i * L, L)
            a = tbuf[0, sl]
            for k in range(1, B_TC):
                a = a + tbuf[k, sl]
            for k in range(NW):
                a = a + buf[k, sl]
            e = jnp.exp(a * 2.0)
            outb[sl] = 1.0 - 2.0 / (e + 1.0)
            return carry

        lax.fori_loop(0, W2 // L, body, 0)
        pltpu.sync_copy(outb, out.at[pl.ds(base, W2)])


def _level_idx(value, low, high, n):
    idx = jnp.round((value - low) / (high - low) * (n - 1)).astype(jnp.int32)
    return jnp.clip(idx, 0, n - 1)


def kernel(input, Wt, Wx, Wy, Wz):
    t = input[:, 0] - input[0, 0]
    xi = _level_idx(input[:, 1], 0.0, 1.0, LEVELS)
    yi = _level_idx(input[:, 2], 0.0, 1.0, LEVELS)
    zi = _level_idx(input[:, 3], 0.0, 1.0, LEVELS)
    ti = _level_idx(t, 0.0, float(TIMESTAMPS), TIMESTAMPS)

    tcp = _tc_partial(
        ti[N_SC:], xi[N_SC:], yi[N_SC:], zi[N_SC:], Wt, Wx, Wy, Wz
    )
    part = _phase1(
        ti[:N_SC].reshape(-1, 1), xi[:N_SC].reshape(-1, 1),
        yi[:N_SC].reshape(-1, 1), zi[:N_SC].reshape(-1, 1),
        Wt, Wx, Wy, Wz,
    )
    return _phase2(part, tcp)


# speedup vs baseline: 2.7564x; 1.0303x over previous
import functools

import jax
import jax.numpy as jnp
from jax import lax
from jax.experimental import pallas as pl
from jax.experimental.pallas import tpu as pltpu
from jax.experimental.pallas import tpu_sc as plsc

LEVELS = 256
TIMESTAMPS = 4096
DIM = 10000
N = 4096

NC = 2
NS = 16
L = 16
NW = NC * NS
N_SC = 3072
N_TC = N - N_SC
SPW = N_SC // NW
CHUNKS = DIM // L

B_TC = 8
G_TC = N_TC // B_TC

W2 = 400
NACT2 = DIM // W2

_MESH = plsc.VectorSubcoreMesh(
    core_axis_name="c", subcore_axis_name="s", num_cores=NC, num_subcores=NS
)


def _worker_id():
    return lax.axis_index("s") * NC + lax.axis_index("c")


@functools.partial(
    pl.kernel,
    out_type=jax.ShapeDtypeStruct((NW, DIM), jnp.float32),
    mesh=_MESH,
    compiler_params=pltpu.CompilerParams(use_tc_tiling_on_sc=False),
    scratch_types=[
        pltpu.VMEM((SPW, 1), jnp.int32),
        pltpu.VMEM((SPW, 1), jnp.int32),
        pltpu.VMEM((SPW, 1), jnp.int32),
        pltpu.VMEM((SPW, 1), jnp.int32),
        pltpu.VMEM((1, DIM), jnp.float32),
        pltpu.VMEM((1, DIM), jnp.float32),
        pltpu.VMEM((1, DIM), jnp.float32),
        pltpu.VMEM((1, DIM), jnp.float32),
        pltpu.VMEM((1, DIM), jnp.float32),
        pltpu.VMEM((1, DIM), jnp.float32),
        pltpu.VMEM((1, DIM), jnp.float32),
        pltpu.VMEM((1, DIM), jnp.float32),
        pltpu.VMEM((DIM,), jnp.float32),
        pltpu.SemaphoreType.DMA,
        pltpu.SemaphoreType.DMA,
    ],
)
def _phase1(ti, xi, yi, zi, Wt, Wx, Wy, Wz, part,
            ti_v, xi_v, yi_v, zi_v,
            wt_a, wx_a, wy_a, wz_a, wt_b, wx_b, wy_b, wz_b,
            acc, sem_a, sem_b):
    wid = _worker_id()
    base = wid * SPW
    pltpu.sync_copy(ti.at[pl.ds(base, SPW)], ti_v)
    pltpu.sync_copy(xi.at[pl.ds(base, SPW)], xi_v)
    pltpu.sync_copy(yi.at[pl.ds(base, SPW)], yi_v)
    pltpu.sync_copy(zi.at[pl.ds(base, SPW)], zi_v)

    @plsc.parallel_loop(0, CHUNKS, unroll=25)
    def _zero(i):
        acc[pl.ds(i * L, L)] = jnp.zeros((L,), jnp.float32)

    set_a = (wt_a, wx_a, wy_a, wz_a)
    set_b = (wt_b, wx_b, wy_b, wz_b)

    def fire(s, bufs, sem):
        pltpu.async_copy(Wt.at[ti_v.at[s]], bufs[0], sem)
        pltpu.async_copy(Wx.at[xi_v.at[s]], bufs[1], sem)
        pltpu.async_copy(Wy.at[yi_v.at[s]], bufs[2], sem)
        pltpu.async_copy(Wz.at[zi_v.at[s]], bufs[3], sem)

    def drain(bufs, sem):
        dummy = Wt.at[pl.ds(0, 1)]
        pltpu.make_async_copy(dummy, bufs[0], sem).wait()
        pltpu.make_async_copy(dummy, bufs[1], sem).wait()
        pltpu.make_async_copy(dummy, bufs[2], sem).wait()
        pltpu.make_async_copy(dummy, bufs[3], sem).wait()

    def accumulate(bufs):
        @plsc.parallel_loop(0, CHUNKS, unroll=25)
        def _chunk(i):
            sl = pl.ds(i * L, L)
            p = bufs[0][0, sl] * bufs[1][0, sl]
            p = p * bufs[2][0, sl]
            p = p * bufs[3][0, sl]
            plsc.addupdate(acc.at[sl], p)

    fire(0, set_a, sem_a)

    def pair_body(p, carry):
        s = 2 * p
        fire(s + 1, set_b, sem_b)
        drain(set_a, sem_a)
        accumulate(set_a)

        @pl.when(p < SPW // 2 - 1)
        def _():
            fire(s + 2, set_a, sem_a)

        drain(set_b, sem_b)
        accumulate(set_b)
        return carry

    lax.fori_loop(0, SPW // 2, pair_body, 0)
    pltpu.sync_copy(acc, part.at[wid])


def _tc_body(ti, xi, yi, zi, wt, wx, wy, wz, o_ref, buf, sems):
    tabs = (wt, wx, wy, wz)
    idxs = (ti, xi, yi, zi)

    def fire(g, slot):
        for k in range(4):
            for j in range(B_TC):
                pltpu.make_async_copy(
                    tabs[k].at[idxs[k][g * B_TC + j]],
                    buf.at[slot, k, j],
                    sems.at[slot, k],
                ).start()

    def drain(slot):
        for k in range(4):
            for j in range(B_TC):
                pltpu.make_async_copy(
                    tabs[k].at[0], buf.at[slot, k, j], sems.at[slot, k]
                ).wait()

    def accumulate(slot):
        p = buf[slot, 0] * buf[slot, 1]
        p = p * (buf[slot, 2] * buf[slot, 3])
        o_ref[...] += p

    o_ref[...] = jnp.zeros_like(o_ref)
    fire(0, 0)

    def pair_body(p, carry):
        g = 2 * p
        fire(g + 1, 1)
        drain(0)
        accumulate(0)

        @pl.when(p < G_TC // 2 - 1)
        def _():
            fire(g + 2, 0)

        drain(1)
        accumulate(1)
        return carry

    lax.fori_loop(0, G_TC // 2, pair_body, 0)


_tc_partial = pl.pallas_call(
    _tc_body,
    grid_spec=pltpu.PrefetchScalarGridSpec(
        num_scalar_prefetch=4,
        grid=(1,),
        in_specs=[
            pl.BlockSpec(memory_space=pl.ANY),
            pl.BlockSpec(memory_space=pl.ANY),
            pl.BlockSpec(memory_space=pl.ANY),
            pl.BlockSpec(memory_space=pl.ANY),
        ],
        out_specs=pl.BlockSpec((B_TC, DIM), lambda i, *_: (0, 0)),
        scratch_shapes=[
            pltpu.VMEM((2, 4, B_TC, DIM), jnp.float32),
            pltpu.SemaphoreType.DMA((2, 4)),
        ],
    ),
    out_shape=jax.ShapeDtypeStruct((B_TC, DIM), jnp.float32),
)


@functools.partial(
    pl.kernel,
    out_type=jax.ShapeDtypeStruct((DIM,), jnp.float32),
    mesh=_MESH,
    compiler_params=pltpu.CompilerParams(use_tc_tiling_on_sc=False),
    scratch_types=[
        pltpu.VMEM((NW, W2), jnp.float32),
        pltpu.VMEM((B_TC, W2), jnp.float32),
        pltpu.VMEM((W2,), jnp.float32),
    ],
)
def _phase2(part, tcp, out, buf, tbuf, outb):
    wid = _worker_id()

    @pl.when(wid < NACT2)
    def _():
        base = wid * W2
        pltpu.sync_copy(part.at[:, pl.ds(base, W2)], buf)
        pltpu.sync_copy(tcp.at[:, pl.ds(base, W2)], tbuf)

        def body(i, carry):
            sl = pl.ds(i * L, L)
            a = tbuf[0, sl]
            for k in range(1, B_TC):
                a = a + tbuf[k, sl]
            for k in range(NW):
                a = a + buf[k, sl]
            e = jnp.exp(a * 2.0)
            outb[sl] = 1.0 - 2.0 / (e + 1.0)
            return carry

        lax.fori_loop(0, W2 // L, body, 0)
        pltpu.sync_copy(outb, out.at[pl.ds(base, W2)])


def _level_idx(value, low, high, n):
    idx = jnp.round((value - low) / (high - low) * (n - 1)).astype(jnp.int32)
    return jnp.clip(idx, 0, n - 1)


def kernel(input, Wt, Wx, Wy, Wz):
    t = input[:, 0] - input[0, 0]
    xi = _level_idx(input[:, 1], 0.0, 1.0, LEVELS)
    yi = _level_idx(input[:, 2], 0.0, 1.0, LEVELS)
    zi = _level_idx(input[:, 3], 0.0, 1.0, LEVELS)
    ti = _level_idx(t, 0.0, float(TIMESTAMPS), TIMESTAMPS)

    tcp = _tc_partial(
        ti[N_SC:], xi[N_SC:], yi[N_SC:], zi[N_SC:], Wt, Wx, Wy, Wz
    )
    part = _phase1(
        ti[:N_SC].reshape(-1, 1), xi[:N_SC].reshape(-1, 1),
        yi[:N_SC].reshape(-1, 1), zi[:N_SC].reshape(-1, 1),
        Wt, Wx, Wy, Wz,
    )
    return _phase2(part, tcp)
